# Initial kernel scaffold; baseline (speedup 1.0000x reference)
import jax
import jax.numpy as jnp
from jax.experimental import pallas as pl


def kernel(x, adj, poly_item, filter_param):
    raise NotImplementedError("write your pallas kernel here")


# speedup vs baseline: 1.6824x; 1.6824x over previous
import math

import jax
import jax.numpy as jnp
from jax.experimental import pallas as pl
from jax.experimental.pallas import tpu as pltpu

_N = 8192
_D = 16
_BM = 1024
_BK = 2048


def _step_kern(a_ref, p_ref, u_ref, s_ref, au_ref, ps_ref):
    j = pl.program_id(1)

    @pl.when(j == 0)
    def _():
        au_ref[...] = jnp.zeros_like(au_ref)
        ps_ref[...] = jnp.zeros_like(ps_ref)

    au_ref[...] += jnp.dot(a_ref[...], u_ref[...],
                           preferred_element_type=jnp.float32)
    ps_ref[...] += jnp.dot(p_ref[...], s_ref[...],
                           preferred_element_type=jnp.float32)


_step = pl.pallas_call(
    _step_kern,
    grid=(_N // _BM, _N // _BK),
    in_specs=[
        pl.BlockSpec((_BM, _BK), lambda i, j: (i, j)),
        pl.BlockSpec((_BM, _BK), lambda i, j: (i, j)),
        pl.BlockSpec((_BK, _D), lambda i, j: (j, 0)),
        pl.BlockSpec((_BK, _D), lambda i, j: (j, 0)),
    ],
    out_specs=[
        pl.BlockSpec((_BM, _D), lambda i, j: (i, 0)),
        pl.BlockSpec((_BM, _D), lambda i, j: (i, 0)),
    ],
    out_shape=[
        jax.ShapeDtypeStruct((_N, _D), jnp.float32),
        jax.ShapeDtypeStruct((_N, _D), jnp.float32),
    ],
    compiler_params=pltpu.CompilerParams(
        dimension_semantics=("parallel", "arbitrary"),
    ),
)


def kernel(x, adj, poly_item, filter_param):
    k = filter_param.shape[0] - 1
    fp = jax.nn.relu(filter_param)[:, 0]
    coefs = [math.comb(k, i) / (2.0 ** k) for i in range(k + 1)]
    a16 = adj.astype(jnp.bfloat16)
    p16 = poly_item.astype(jnp.bfloat16)
    u = x
    s = coefs[k] * fp[k] * x
    for t in range(1, k + 1):
        au, ps = _step(a16, p16,
                       u.astype(jnp.bfloat16), s.astype(jnp.bfloat16))
        u = au
        s = coefs[k - t] * fp[k - t] * au + ps
    return s


# speedup vs baseline: 1.9404x; 1.1533x over previous
import math

import jax
import jax.numpy as jnp
from jax.experimental import pallas as pl
from jax.experimental.pallas import tpu as pltpu

_N = 8192
_D = 16
_BM = 1024
_BK = 2048
_BK1 = 1024


def _step_kern(a_ref, p_ref, u_ref, s_ref, au_ref, ps_ref):
    j = pl.program_id(1)

    @pl.when(j == 0)
    def _():
        au_ref[...] = jnp.zeros_like(au_ref)
        ps_ref[...] = jnp.zeros_like(ps_ref)

    au_ref[...] += jnp.dot(a_ref[...], u_ref[...],
                           preferred_element_type=jnp.float32)
    ps_ref[...] += jnp.dot(p_ref[...], s_ref[...],
                           preferred_element_type=jnp.float32)


def _step1_kern(a_ref, p_ref, u_ref, s_ref, au_ref, ps_ref, a16_ref, p16_ref):
    j = pl.program_id(1)

    @pl.when(j == 0)
    def _():
        au_ref[...] = jnp.zeros_like(au_ref)
        ps_ref[...] = jnp.zeros_like(ps_ref)

    a16 = a_ref[...].astype(jnp.bfloat16)
    p16 = p_ref[...].astype(jnp.bfloat16)
    a16_ref[...] = a16
    p16_ref[...] = p16
    au_ref[...] += jnp.dot(a16, u_ref[...],
                           preferred_element_type=jnp.float32)
    ps_ref[...] += jnp.dot(p16, s_ref[...],
                           preferred_element_type=jnp.float32)


_step = pl.pallas_call(
    _step_kern,
    grid=(_N // _BM, _N // _BK),
    in_specs=[
        pl.BlockSpec((_BM, _BK), lambda i, j: (i, j)),
        pl.BlockSpec((_BM, _BK), lambda i, j: (i, j)),
        pl.BlockSpec((_BK, _D), lambda i, j: (j, 0)),
        pl.BlockSpec((_BK, _D), lambda i, j: (j, 0)),
    ],
    out_specs=[
        pl.BlockSpec((_BM, _D), lambda i, j: (i, 0)),
        pl.BlockSpec((_BM, _D), lambda i, j: (i, 0)),
    ],
    out_shape=[
        jax.ShapeDtypeStruct((_N, _D), jnp.float32),
        jax.ShapeDtypeStruct((_N, _D), jnp.float32),
    ],
    compiler_params=pltpu.CompilerParams(
        dimension_semantics=("parallel", "arbitrary"),
    ),
)

_step1 = pl.pallas_call(
    _step1_kern,
    grid=(_N // _BM, _N // _BK1),
    in_specs=[
        pl.BlockSpec((_BM, _BK1), lambda i, j: (i, j)),
        pl.BlockSpec((_BM, _BK1), lambda i, j: (i, j)),
        pl.BlockSpec((_BK1, _D), lambda i, j: (j, 0)),
        pl.BlockSpec((_BK1, _D), lambda i, j: (j, 0)),
    ],
    out_specs=[
        pl.BlockSpec((_BM, _D), lambda i, j: (i, 0)),
        pl.BlockSpec((_BM, _D), lambda i, j: (i, 0)),
        pl.BlockSpec((_BM, _BK1), lambda i, j: (i, j)),
        pl.BlockSpec((_BM, _BK1), lambda i, j: (i, j)),
    ],
    out_shape=[
        jax.ShapeDtypeStruct((_N, _D), jnp.float32),
        jax.ShapeDtypeStruct((_N, _D), jnp.float32),
        jax.ShapeDtypeStruct((_N, _N), jnp.bfloat16),
        jax.ShapeDtypeStruct((_N, _N), jnp.bfloat16),
    ],
    compiler_params=pltpu.CompilerParams(
        dimension_semantics=("parallel", "arbitrary"),
    ),
)


def kernel(x, adj, poly_item, filter_param):
    k = filter_param.shape[0] - 1
    fp = jax.nn.relu(filter_param)[:, 0]
    coefs = [math.comb(k, i) / (2.0 ** k) for i in range(k + 1)]
    u = x
    s = coefs[k] * fp[k] * x
    au, ps, a16, p16 = _step1(adj, poly_item,
                              u.astype(jnp.bfloat16), s.astype(jnp.bfloat16))
    u = au
    s = coefs[k - 1] * fp[k - 1] * au + ps
    for t in range(2, k + 1):
        au, ps = _step(a16, p16,
                       u.astype(jnp.bfloat16), s.astype(jnp.bfloat16))
        u = au
        s = coefs[k - t] * fp[k - t] * au + ps
    return s
